# Initial kernel scaffold; baseline (speedup 1.0000x reference)
import jax
import jax.numpy as jnp
from jax.experimental import pallas as pl


def kernel(x, centroids):
    raise NotImplementedError("write your pallas kernel here")


# speedup vs baseline: 1.7311x; 1.7311x over previous
import functools

import jax
import jax.numpy as jnp
from jax.experimental import pallas as pl

_NUM_CLUSTERS = 8
_EPS = 1e-05
_BLOCK_ROWS = 512


def _relu6(v):
    return jnp.clip(v, 0.0, 6.0)


def _elu(v):
    return jnp.where(v > 0, v, jnp.exp(jnp.minimum(v, 0.0)) - 1.0)


def _softplus(v):
    return jnp.maximum(v, 0.0) + jnp.log(1.0 + jnp.exp(-jnp.abs(v)))


_ACTS = [
    jax.nn.relu,
    jax.nn.gelu,
    jnp.tanh,
    jax.nn.silu,
    jax.nn.sigmoid,
    _relu6,
    _elu,
    _softplus,
]


def _body(x_ref, c_ref, o_ref):
    xb = x_ref[...]
    cb = c_ref[...]
    d = xb.shape[1]

    dots = jax.lax.dot_general(
        xb, cb, (((1,), (1,)), ((), ())), preferred_element_type=jnp.float32
    )
    c2 = jnp.sum(cb * cb, axis=1, keepdims=True)
    x2 = jnp.sum(xb * xb, axis=1, keepdims=True)
    dist = x2 - 2.0 * dots + jnp.transpose(c2)

    mind = jnp.min(dist, axis=1, keepdims=True)
    lane = jax.lax.broadcasted_iota(jnp.int32, dist.shape, 1)
    labels = jnp.min(
        jnp.where(dist == mind, lane, _NUM_CLUSTERS), axis=1, keepdims=True
    )

    mean = jnp.sum(xb, axis=1, keepdims=True) * (1.0 / d)
    cen = xb - mean
    var = jnp.sum(cen * cen, axis=1, keepdims=True) * (1.0 / (d - 1))
    xn = cen / jnp.sqrt(var + _EPS)

    out = _ACTS[0](xn)
    for c in range(1, _NUM_CLUSTERS):
        out = jnp.where(labels == c, _ACTS[c](xn), out)
    o_ref[...] = out


@jax.jit
def kernel(x, centroids):
    n, d = x.shape
    grid = (n // _BLOCK_ROWS,)
    return pl.pallas_call(
        _body,
        grid=grid,
        in_specs=[
            pl.BlockSpec((_BLOCK_ROWS, d), lambda i: (i, 0)),
            pl.BlockSpec((_NUM_CLUSTERS, d), lambda i: (0, 0)),
        ],
        out_specs=pl.BlockSpec((_BLOCK_ROWS, d), lambda i: (i, 0)),
        out_shape=jax.ShapeDtypeStruct((n, d), x.dtype),
    )(x, centroids)


# speedup vs baseline: 2.3658x; 1.3667x over previous
import jax
import jax.numpy as jnp
from jax.experimental import pallas as pl

_NUM_CLUSTERS = 8
_EPS = 1e-05
_BLOCK_ROWS = 512

_GELU_A = 2.0 * 0.7978845608028654
_GELU_B = _GELU_A * 0.044715


def _body(x_ref, c_ref, o_ref):
    xb = x_ref[...]
    cb = c_ref[...]
    d = xb.shape[1]

    s1 = jnp.sum(xb, axis=1, keepdims=True)
    s2 = jnp.sum(xb * xb, axis=1, keepdims=True)

    dots = jax.lax.dot_general(
        xb, cb, (((1,), (1,)), ((), ())), preferred_element_type=jnp.float32
    )
    c2 = jnp.sum(cb * cb, axis=1, keepdims=True)
    dist = s2 - 2.0 * dots + jnp.transpose(c2)

    mind = jnp.min(dist, axis=1, keepdims=True)
    lane = jax.lax.broadcasted_iota(jnp.int32, dist.shape, 1)
    lab = jnp.min(
        jnp.where(dist == mind, lane, _NUM_CLUSTERS), axis=1, keepdims=True
    )

    mean = s1 * (1.0 / d)
    var = (s2 - s1 * mean) * (1.0 / (d - 1))
    rstd = jax.lax.rsqrt(var + _EPS)
    xn = (xb - mean) * rstd

    fa = jnp.where(
        lab == 1, -_GELU_A,
        jnp.where(
            lab == 2, -2.0,
            jnp.where((lab == 3) | (lab == 4), -1.0,
                      jnp.where(lab >= 6, 1.0, 0.0)),
        ),
    )
    fb = jnp.where(lab == 1, -_GELU_B, 0.0)
    fp = jnp.where((lab == 1) | (lab == 3), 1.0, 0.0)
    fq = jnp.where(lab == 2, 2.0, jnp.where(lab == 4, 1.0, 0.0))
    ft = jnp.where(lab == 2, -1.0, 0.0)
    upper = jnp.where(lab == 5, 6.0, 3.0e38)

    xn3 = (xn * xn) * xn
    em = jnp.exp(fa * xn + fb * xn3)
    t1 = 1.0 + em
    s = 1.0 / t1
    sig_out = (fp * xn + fq) * s + ft

    relu = jnp.maximum(xn, 0.0)
    pwl = jnp.minimum(relu, upper)

    elu_out = jnp.where(xn > 0.0, xn, em - 1.0)
    sp_out = jnp.log(t1)

    is_sig = (lab >= 1) & (lab <= 4)
    out = jnp.where(is_sig, sig_out, pwl)
    out = jnp.where(lab == 6, elu_out, out)
    out = jnp.where(lab == 7, sp_out, out)
    o_ref[...] = out


@jax.jit
def kernel(x, centroids):
    n, d = x.shape
    grid = (n // _BLOCK_ROWS,)
    return pl.pallas_call(
        _body,
        grid=grid,
        in_specs=[
            pl.BlockSpec((_BLOCK_ROWS, d), lambda i: (i, 0)),
            pl.BlockSpec((_NUM_CLUSTERS, d), lambda i: (0, 0)),
        ],
        out_specs=pl.BlockSpec((_BLOCK_ROWS, d), lambda i: (i, 0)),
        out_shape=jax.ShapeDtypeStruct((n, d), x.dtype),
    )(x, centroids)
